# baseline (speedup 1.0000x reference)
import functools

import jax
import jax.numpy as jnp
from jax import lax
from jax.experimental import pallas as pl
from jax.experimental.pallas import tpu as pltpu
from jax.experimental.pallas import tpu_sc as plsc

N = 10000
E = 320000
D = 128
BN_EPS = 1e-5

NC = 2
NS = 16
CHUNK = 128
STAGES = 5
CPS = 16
NCHUNKS = STAGES * CPS
E_PAD = NC * NS * NCHUNKS * CHUNK
N_ACC = N + 16
DUMMY_DST = N + 8
ROWS_PER_TILE = 624
TAIL_ROWS = N - NS * ROWS_PER_TILE


def _sc_segment_sum(x, src, dst):
    mesh = plsc.VectorSubcoreMesh(core_axis_name="c", subcore_axis_name="s")

    @functools.partial(
        pl.kernel,
        out_type=jax.ShapeDtypeStruct((NC, N, D), jnp.float32),
        mesh=mesh,
        scratch_types=[
            pltpu.VMEM((CPS, CHUNK), jnp.int32),
            pltpu.VMEM((CPS, CHUNK), jnp.int32),
            pltpu.VMEM((CHUNK, D), jnp.float32),
            pltpu.VMEM((CHUNK, D), jnp.float32),
            pltpu.VMEM_SHARED((N_ACC, D), jnp.float32),
            pltpu.SemaphoreType.DMA,
            pltpu.SemaphoreType.DMA,
            pltpu.SemaphoreType.DMA,
        ],
    )
    def sc_kernel(x_hbm, src_hbm, dst_hbm, out_hbm, src_v, dst_v, rows0,
                  rows1, acc, gsem0, gsem1, isem):
        cid = lax.axis_index("c")
        sid = lax.axis_index("s")
        row0 = sid * ROWS_PER_TILE

        def start_gather(j, buf, sem):
            pltpu.make_async_copy(x_hbm.at[src_v.at[j]], buf, sem).start()

        def wait_gather(j, buf, sem):
            pltpu.make_async_copy(x_hbm.at[src_v.at[j]], buf, sem).wait()

        pltpu.sync_copy(src_hbm.at[cid, sid, 0], src_v)
        pltpu.sync_copy(dst_hbm.at[cid, sid, 0], dst_v)
        start_gather(0, rows0, gsem0)
        start_gather(1, rows1, gsem1)

        pltpu.make_async_copy(x_hbm.at[pl.ds(row0, ROWS_PER_TILE)],
                              acc.at[pl.ds(row0, ROWS_PER_TILE)],
                              isem).start()

        @pl.when(sid == 0)
        def _():
            pltpu.sync_copy(x_hbm.at[pl.ds(NS * ROWS_PER_TILE, TAIL_ROWS)],
                            acc.at[pl.ds(NS * ROWS_PER_TILE, TAIL_ROWS)])

        pltpu.make_async_copy(x_hbm.at[pl.ds(row0, ROWS_PER_TILE)],
                              acc.at[pl.ds(row0, ROWS_PER_TILE)],
                              isem).wait()
        plsc.subcore_barrier()

        @pl.loop(0, STAGES)
        def _(s):
            @pl.when(s > 0)
            def _():
                pltpu.sync_copy(src_hbm.at[cid, sid, s], src_v)
                pltpu.sync_copy(dst_hbm.at[cid, sid, s], dst_v)
                start_gather(0, rows0, gsem0)
                start_gather(1, rows1, gsem1)

            @pl.loop(0, CPS // 2)
            def _(j):
                wait_gather(2 * j, rows0, gsem0)
                pltpu.sync_copy(rows0, acc.at[dst_v.at[2 * j]], add=True)

                @pl.when(j < CPS // 2 - 1)
                def _():
                    start_gather(2 * j + 2, rows0, gsem0)

                wait_gather(2 * j + 1, rows1, gsem1)
                pltpu.sync_copy(rows1, acc.at[dst_v.at[2 * j + 1]], add=True)

                @pl.when(j < CPS // 2 - 1)
                def _():
                    start_gather(2 * j + 3, rows1, gsem1)

        plsc.subcore_barrier()

        pltpu.sync_copy(acc.at[pl.ds(row0, ROWS_PER_TILE)],
                        out_hbm.at[cid].at[pl.ds(row0, ROWS_PER_TILE)])

        @pl.when(sid == 0)
        def _():
            pltpu.sync_copy(acc.at[pl.ds(NS * ROWS_PER_TILE, TAIL_ROWS)],
                            out_hbm.at[cid].at[pl.ds(NS * ROWS_PER_TILE,
                                                     TAIL_ROWS)])

    return sc_kernel(x, src, dst)


def _mlp_body(parts, x, w1, b1, g1, bt1, w2, b2, g2, bt2, o):
    h = parts[0] + parts[1] - x[...]
    y = lax.dot_general(h, w1[...], (((1,), (1,)), ((), ())),
                        preferred_element_type=jnp.float32) + b1[...]
    m = jnp.mean(y, axis=0, keepdims=True)
    v = jnp.mean((y - m) * (y - m), axis=0, keepdims=True)
    y = (y - m) * lax.rsqrt(v + BN_EPS) * g1[...] + bt1[...]
    y = jnp.maximum(y, 0.0)
    z = lax.dot_general(y, w2[...], (((1,), (1,)), ((), ())),
                        preferred_element_type=jnp.float32) + b2[...]
    m2 = jnp.mean(z, axis=0, keepdims=True)
    v2 = jnp.mean((z - m2) * (z - m2), axis=0, keepdims=True)
    z = (z - m2) * lax.rsqrt(v2 + BN_EPS) * g2[...] + bt2[...]
    o[...] = jnp.maximum(z, 0.0)


@jax.jit
def kernel(x, edge_index, W1, b1, g1, bt1, W2, b2, g2, bt2):
    ei = edge_index.astype(jnp.int32)
    src = jnp.concatenate(
        [ei[0], jnp.zeros((E_PAD - E,), jnp.int32)]
    ).reshape(NC, NS, STAGES, CPS, CHUNK)
    dst = jnp.concatenate(
        [ei[1], jnp.full((E_PAD - E,), DUMMY_DST, jnp.int32)]
    ).reshape(NC, NS, STAGES, CPS, CHUNK)

    parts = _sc_segment_sum(x, src, dst)

    row = lambda v: v.reshape(1, D)
    return pl.pallas_call(
        _mlp_body,
        out_shape=jax.ShapeDtypeStruct((N, D), jnp.float32),
    )(parts, x, W1, row(b1), row(g1), row(bt1),
      W2, row(b2), row(g2), row(bt2))


# speedup vs baseline: 1.1756x; 1.1756x over previous
import functools

import jax
import jax.numpy as jnp
from jax import lax
from jax.experimental import pallas as pl
from jax.experimental.pallas import tpu as pltpu
from jax.experimental.pallas import tpu_sc as plsc

N = 10000
E = 320000
D = 128
BN_EPS = 1e-5

NC = 2
NS = 16
CHUNK = 128
STAGES = 5
CPS = 16
NCHUNKS = STAGES * CPS
EPT = E // (NC * NS)
PAD_PER_TILE = NCHUNKS * CHUNK - EPT
N_ACC = N + 16
ROWS_PER_TILE = 624
TAIL_ROWS = N - NS * ROWS_PER_TILE


def _sc_segment_sum(x, src, dst):
    mesh = plsc.VectorSubcoreMesh(core_axis_name="c", subcore_axis_name="s")

    @functools.partial(
        pl.kernel,
        out_type=jax.ShapeDtypeStruct((NC, N, D), jnp.float32),
        mesh=mesh,
        scratch_types=[
            pltpu.VMEM((CPS, CHUNK), jnp.int32),
            pltpu.VMEM((CPS, CHUNK), jnp.int32),
            pltpu.VMEM((CHUNK, D), jnp.float32),
            pltpu.VMEM((CHUNK, D), jnp.float32),
            pltpu.VMEM_SHARED((N_ACC, D), jnp.float32),
            pltpu.SemaphoreType.DMA,
            pltpu.SemaphoreType.DMA,
            pltpu.SemaphoreType.DMA,
        ],
    )
    def sc_kernel(x_hbm, src_hbm, dst_hbm, out_hbm, src_v, dst_v, rows0,
                  rows1, acc, gsem0, gsem1, isem):
        cid = lax.axis_index("c")
        sid = lax.axis_index("s")
        row0 = sid * ROWS_PER_TILE

        def start_gather(j, buf, sem):
            pltpu.make_async_copy(x_hbm.at[src_v.at[j]], buf, sem).start()

        def wait_gather(j, buf, sem):
            pltpu.make_async_copy(x_hbm.at[src_v.at[j]], buf, sem).wait()

        pltpu.sync_copy(src_hbm.at[cid, sid, 0], src_v)
        pltpu.sync_copy(dst_hbm.at[cid, sid, 0], dst_v)
        start_gather(0, rows0, gsem0)
        start_gather(1, rows1, gsem1)

        pltpu.make_async_copy(x_hbm.at[pl.ds(row0, ROWS_PER_TILE)],
                              acc.at[pl.ds(row0, ROWS_PER_TILE)],
                              isem).start()

        @pl.when(sid == 0)
        def _():
            pltpu.sync_copy(x_hbm.at[pl.ds(NS * ROWS_PER_TILE, TAIL_ROWS)],
                            acc.at[pl.ds(NS * ROWS_PER_TILE, TAIL_ROWS)])

        pltpu.make_async_copy(x_hbm.at[pl.ds(row0, ROWS_PER_TILE)],
                              acc.at[pl.ds(row0, ROWS_PER_TILE)],
                              isem).wait()
        plsc.subcore_barrier()

        @pl.loop(0, STAGES)
        def _(s):
            @pl.when(s > 0)
            def _():
                pltpu.sync_copy(src_hbm.at[cid, sid, s], src_v)
                pltpu.sync_copy(dst_hbm.at[cid, sid, s], dst_v)
                start_gather(0, rows0, gsem0)
                start_gather(1, rows1, gsem1)

            @pl.loop(0, CPS // 2)
            def _(j):
                wait_gather(2 * j, rows0, gsem0)
                pltpu.sync_copy(rows0, acc.at[dst_v.at[2 * j]], add=True)

                @pl.when(j < CPS // 2 - 1)
                def _():
                    start_gather(2 * j + 2, rows0, gsem0)

                wait_gather(2 * j + 1, rows1, gsem1)
                pltpu.sync_copy(rows1, acc.at[dst_v.at[2 * j + 1]], add=True)

                @pl.when(j < CPS // 2 - 1)
                def _():
                    start_gather(2 * j + 3, rows1, gsem1)

        plsc.subcore_barrier()

        pltpu.sync_copy(acc.at[pl.ds(row0, ROWS_PER_TILE)],
                        out_hbm.at[cid].at[pl.ds(row0, ROWS_PER_TILE)])

        @pl.when(sid == 0)
        def _():
            pltpu.sync_copy(acc.at[pl.ds(NS * ROWS_PER_TILE, TAIL_ROWS)],
                            out_hbm.at[cid].at[pl.ds(NS * ROWS_PER_TILE,
                                                     TAIL_ROWS)])

    return sc_kernel(x, src, dst)


def _mlp_body(parts, x, w1, b1, g1, bt1, w2, b2, g2, bt2, o):
    h = parts[0] + parts[1] - x[...]
    y = lax.dot_general(h, w1[...], (((1,), (1,)), ((), ())),
                        preferred_element_type=jnp.float32) + b1[...]
    m = jnp.mean(y, axis=0, keepdims=True)
    v = jnp.mean((y - m) * (y - m), axis=0, keepdims=True)
    y = (y - m) * lax.rsqrt(v + BN_EPS) * g1[...] + bt1[...]
    y = jnp.maximum(y, 0.0)
    z = lax.dot_general(y, w2[...], (((1,), (1,)), ((), ())),
                        preferred_element_type=jnp.float32) + b2[...]
    m2 = jnp.mean(z, axis=0, keepdims=True)
    v2 = jnp.mean((z - m2) * (z - m2), axis=0, keepdims=True)
    z = (z - m2) * lax.rsqrt(v2 + BN_EPS) * g2[...] + bt2[...]
    o[...] = jnp.maximum(z, 0.0)


@jax.jit
def kernel(x, edge_index, W1, b1, g1, bt1, W2, b2, g2, bt2):
    ei = edge_index.astype(jnp.int32)
    pad_src = jnp.zeros((NC, NS, PAD_PER_TILE), jnp.int32)
    pad_dst = jnp.broadcast_to(
        N + (jnp.arange(PAD_PER_TILE, dtype=jnp.int32) % 16),
        (NC, NS, PAD_PER_TILE))
    src = jnp.concatenate([ei[0].reshape(NC, NS, EPT), pad_src],
                          axis=-1).reshape(NC, NS, STAGES, CPS, CHUNK)
    dst = jnp.concatenate([ei[1].reshape(NC, NS, EPT), pad_dst],
                          axis=-1).reshape(NC, NS, STAGES, CPS, CHUNK)

    parts = _sc_segment_sum(x, src, dst)

    row = lambda v: v.reshape(1, D)
    return pl.pallas_call(
        _mlp_body,
        out_shape=jax.ShapeDtypeStruct((N, D), jnp.float32),
    )(parts, x, W1, row(b1), row(g1), row(bt1),
      W2, row(b2), row(g2), row(bt2))


# speedup vs baseline: 3.1823x; 2.7068x over previous
import functools

import jax
import jax.numpy as jnp
from jax import lax
from jax.experimental import pallas as pl
from jax.experimental.pallas import tpu as pltpu
from jax.experimental.pallas import tpu_sc as plsc

N = 10000
E = 320000
D = 128
BN_EPS = 1e-5

NC = 2
NS = 16
CHUNK = 125
NCHUNKS = E // (NC * NS) // CHUNK
STAGES = 5
CPS = NCHUNKS // STAGES
N_ACC = N
ROWS_PER_TILE = 624
TAIL_ROWS = N - NS * ROWS_PER_TILE


def _sc_segment_sum(x, src, dst):
    mesh = plsc.VectorSubcoreMesh(core_axis_name="c", subcore_axis_name="s")

    @functools.partial(
        pl.kernel,
        out_type=jax.ShapeDtypeStruct((NC, N, D), jnp.float32),
        mesh=mesh,
        scratch_types=[
            pltpu.VMEM((CPS, CHUNK), jnp.int32),
            pltpu.VMEM((CPS, CHUNK), jnp.int32),
            pltpu.VMEM((CHUNK, D), jnp.float32),
            pltpu.VMEM((CHUNK, D), jnp.float32),
            pltpu.VMEM_SHARED((N_ACC, D), jnp.float32),
            pltpu.SemaphoreType.DMA,
            pltpu.SemaphoreType.DMA,
            pltpu.SemaphoreType.DMA,
        ],
    )
    def sc_kernel(x_hbm, src_hbm, dst_hbm, out_hbm, src_v, dst_v, rows0,
                  rows1, acc, gsem0, gsem1, isem):
        cid = lax.axis_index("c")
        sid = lax.axis_index("s")
        row0 = sid * ROWS_PER_TILE

        def start_gather(j, buf, sem):
            pltpu.make_async_copy(x_hbm.at[src_v.at[j]], buf, sem).start()

        def wait_gather(j, buf, sem):
            pltpu.make_async_copy(x_hbm.at[src_v.at[j]], buf, sem).wait()

        pltpu.sync_copy(src_hbm.at[cid, sid, 0], src_v)
        pltpu.sync_copy(dst_hbm.at[cid, sid, 0], dst_v)
        start_gather(0, rows0, gsem0)
        start_gather(1, rows1, gsem1)

        pltpu.make_async_copy(x_hbm.at[pl.ds(row0, ROWS_PER_TILE)],
                              acc.at[pl.ds(row0, ROWS_PER_TILE)],
                              isem).start()

        @pl.when(sid == 0)
        def _():
            pltpu.sync_copy(x_hbm.at[pl.ds(NS * ROWS_PER_TILE, TAIL_ROWS)],
                            acc.at[pl.ds(NS * ROWS_PER_TILE, TAIL_ROWS)])

        pltpu.make_async_copy(x_hbm.at[pl.ds(row0, ROWS_PER_TILE)],
                              acc.at[pl.ds(row0, ROWS_PER_TILE)],
                              isem).wait()
        plsc.subcore_barrier()

        @pl.loop(0, STAGES)
        def _(s):
            @pl.when(s > 0)
            def _():
                pltpu.sync_copy(src_hbm.at[cid, sid, s], src_v)
                pltpu.sync_copy(dst_hbm.at[cid, sid, s], dst_v)
                start_gather(0, rows0, gsem0)
                start_gather(1, rows1, gsem1)

            @pl.loop(0, CPS // 2)
            def _(j):
                wait_gather(2 * j, rows0, gsem0)
                pltpu.sync_copy(rows0, acc.at[dst_v.at[2 * j]], add=True)

                @pl.when(j < CPS // 2 - 1)
                def _():
                    start_gather(2 * j + 2, rows0, gsem0)

                wait_gather(2 * j + 1, rows1, gsem1)
                pltpu.sync_copy(rows1, acc.at[dst_v.at[2 * j + 1]], add=True)

                @pl.when(j < CPS // 2 - 1)
                def _():
                    start_gather(2 * j + 3, rows1, gsem1)

        plsc.subcore_barrier()

        pltpu.sync_copy(acc.at[pl.ds(row0, ROWS_PER_TILE)],
                        out_hbm.at[cid].at[pl.ds(row0, ROWS_PER_TILE)])

        @pl.when(sid == 0)
        def _():
            pltpu.sync_copy(acc.at[pl.ds(NS * ROWS_PER_TILE, TAIL_ROWS)],
                            out_hbm.at[cid].at[pl.ds(NS * ROWS_PER_TILE,
                                                     TAIL_ROWS)])

    return sc_kernel(x, src, dst)


def _mlp_body(parts, x, w1, b1, g1, bt1, w2, b2, g2, bt2, o):
    h = parts[0] + parts[1] - x[...]
    y = lax.dot_general(h, w1[...], (((1,), (1,)), ((), ())),
                        preferred_element_type=jnp.float32) + b1[...]
    m = jnp.mean(y, axis=0, keepdims=True)
    v = jnp.mean((y - m) * (y - m), axis=0, keepdims=True)
    y = (y - m) * lax.rsqrt(v + BN_EPS) * g1[...] + bt1[...]
    y = jnp.maximum(y, 0.0)
    z = lax.dot_general(y, w2[...], (((1,), (1,)), ((), ())),
                        preferred_element_type=jnp.float32) + b2[...]
    m2 = jnp.mean(z, axis=0, keepdims=True)
    v2 = jnp.mean((z - m2) * (z - m2), axis=0, keepdims=True)
    z = (z - m2) * lax.rsqrt(v2 + BN_EPS) * g2[...] + bt2[...]
    o[...] = jnp.maximum(z, 0.0)


@jax.jit
def kernel(x, edge_index, W1, b1, g1, bt1, W2, b2, g2, bt2):
    ei = edge_index.astype(jnp.int32)
    src = ei[0].reshape(NC, NS, STAGES, CPS, CHUNK)
    dst = ei[1].reshape(NC, NS, STAGES, CPS, CHUNK)

    parts = _sc_segment_sum(x, src, dst)

    row = lambda v: v.reshape(1, D)
    return pl.pallas_call(
        _mlp_body,
        out_shape=jax.ShapeDtypeStruct((N, D), jnp.float32),
    )(parts, x, W1, row(b1), row(g1), row(bt1),
      W2, row(b2), row(g2), row(bt2))


# speedup vs baseline: 3.4115x; 1.0720x over previous
import functools

import jax
import jax.numpy as jnp
from jax import lax
from jax.experimental import pallas as pl
from jax.experimental.pallas import tpu as pltpu
from jax.experimental.pallas import tpu_sc as plsc

N = 10000
E = 320000
D = 128
BN_EPS = 1e-5

NC = 2
NS = 16
CHUNK = 125
NCHUNKS = E // (NC * NS) // CHUNK
STAGES = 5
CPS = NCHUNKS // STAGES
N_ACC = N
ROWS_PER_TILE = 624
TAIL_ROWS = N - NS * ROWS_PER_TILE


def _sc_segment_sum(x, ed):
    mesh = plsc.VectorSubcoreMesh(core_axis_name="c", subcore_axis_name="s")

    @functools.partial(
        pl.kernel,
        out_type=jax.ShapeDtypeStruct((NC, N, D), jnp.float32),
        mesh=mesh,
        scratch_types=[
            pltpu.VMEM((CPS, CHUNK), jnp.int32),
            pltpu.VMEM((CPS, CHUNK), jnp.int32),
            pltpu.VMEM((CHUNK, D), jnp.float32),
            pltpu.VMEM((CHUNK, D), jnp.float32),
            pltpu.VMEM_SHARED((N_ACC, D), jnp.float32),
            pltpu.SemaphoreType.DMA,
            pltpu.SemaphoreType.DMA,
            pltpu.SemaphoreType.DMA,
        ],
    )
    def sc_kernel(x_hbm, ed_hbm, out_hbm, src_v, dst_v, rows0,
                  rows1, acc, gsem0, gsem1, isem):
        cid = lax.axis_index("c")
        sid = lax.axis_index("s")
        row0 = sid * ROWS_PER_TILE

        def start_gather(j, buf, sem):
            pltpu.make_async_copy(x_hbm.at[src_v.at[j]], buf, sem).start()

        def wait_gather(j, buf, sem):
            pltpu.make_async_copy(x_hbm.at[src_v.at[j]], buf, sem).wait()

        pltpu.sync_copy(ed_hbm.at[0, cid, sid, 0], src_v)
        pltpu.sync_copy(ed_hbm.at[1, cid, sid, 0], dst_v)
        start_gather(0, rows0, gsem0)
        start_gather(1, rows1, gsem1)

        pltpu.make_async_copy(x_hbm.at[pl.ds(row0, ROWS_PER_TILE)],
                              acc.at[pl.ds(row0, ROWS_PER_TILE)],
                              isem).start()

        @pl.when(sid == 0)
        def _():
            pltpu.sync_copy(x_hbm.at[pl.ds(NS * ROWS_PER_TILE, TAIL_ROWS)],
                            acc.at[pl.ds(NS * ROWS_PER_TILE, TAIL_ROWS)])

        pltpu.make_async_copy(x_hbm.at[pl.ds(row0, ROWS_PER_TILE)],
                              acc.at[pl.ds(row0, ROWS_PER_TILE)],
                              isem).wait()
        plsc.subcore_barrier()

        @pl.loop(0, STAGES)
        def _(s):
            @pl.when(s > 0)
            def _():
                pltpu.sync_copy(ed_hbm.at[0, cid, sid, s], src_v)
                pltpu.sync_copy(ed_hbm.at[1, cid, sid, s], dst_v)
                start_gather(0, rows0, gsem0)
                start_gather(1, rows1, gsem1)

            @pl.loop(0, CPS // 2)
            def _(j):
                wait_gather(2 * j, rows0, gsem0)
                pltpu.sync_copy(rows0, acc.at[dst_v.at[2 * j]], add=True)

                @pl.when(j < CPS // 2 - 1)
                def _():
                    start_gather(2 * j + 2, rows0, gsem0)

                wait_gather(2 * j + 1, rows1, gsem1)
                pltpu.sync_copy(rows1, acc.at[dst_v.at[2 * j + 1]], add=True)

                @pl.when(j < CPS // 2 - 1)
                def _():
                    start_gather(2 * j + 3, rows1, gsem1)

        plsc.subcore_barrier()

        pltpu.sync_copy(acc.at[pl.ds(row0, ROWS_PER_TILE)],
                        out_hbm.at[cid].at[pl.ds(row0, ROWS_PER_TILE)])

        @pl.when(sid == 0)
        def _():
            pltpu.sync_copy(acc.at[pl.ds(NS * ROWS_PER_TILE, TAIL_ROWS)],
                            out_hbm.at[cid].at[pl.ds(NS * ROWS_PER_TILE,
                                                     TAIL_ROWS)])

    return sc_kernel(x, ed)


def _mlp_body(parts, x, w1, b1, g1, bt1, w2, b2, g2, bt2, o):
    h = parts[0] + parts[1] - x[...]
    y = lax.dot_general(h, w1[...], (((1,), (1,)), ((), ())),
                        preferred_element_type=jnp.float32) + b1[...]
    m = jnp.mean(y, axis=0, keepdims=True)
    v = jnp.mean((y - m) * (y - m), axis=0, keepdims=True)
    y = (y - m) * lax.rsqrt(v + BN_EPS) * g1[...] + bt1[...]
    y = jnp.maximum(y, 0.0)
    z = lax.dot_general(y, w2[...], (((1,), (1,)), ((), ())),
                        preferred_element_type=jnp.float32) + b2[...]
    m2 = jnp.mean(z, axis=0, keepdims=True)
    v2 = jnp.mean((z - m2) * (z - m2), axis=0, keepdims=True)
    z = (z - m2) * lax.rsqrt(v2 + BN_EPS) * g2[...] + bt2[...]
    o[...] = jnp.maximum(z, 0.0)


@jax.jit
def kernel(x, edge_index, W1, b1, g1, bt1, W2, b2, g2, bt2):
    ed = edge_index.astype(jnp.int32).reshape(2, NC, NS, STAGES, CPS, CHUNK)

    parts = _sc_segment_sum(x, ed)

    row = lambda v: v.reshape(1, D)
    return pl.pallas_call(
        _mlp_body,
        out_shape=jax.ShapeDtypeStruct((N, D), jnp.float32),
    )(parts, x, W1, row(b1), row(g1), row(bt1),
      W2, row(b2), row(g2), row(bt2))
